# Initial kernel scaffold; baseline (speedup 1.0000x reference)
import jax
import jax.numpy as jnp
from jax.experimental import pallas as pl


def kernel(x, edge_index, edge_attr, batch, n_graphs, lin_in_W, lin_in_b, msg_W1, msg_b1, msg_W2, msg_b2, bn_gamma, bn_beta, bn_mean, bn_var, gru_Wih, gru_Whh, gru_bih, gru_bhh, ro_W, ro_b):
    raise NotImplementedError("write your pallas kernel here")


# speedup vs baseline: 2.8175x; 2.8175x over previous
import jax
import jax.numpy as jnp
from jax import lax
from jax.experimental import pallas as pl
from jax.experimental.pallas import tpu as pltpu
from jax.experimental.pallas import tpu_sc as plsc

N = 10000
E = 320000
H = 128
L = 3
NG = 256
EDGE_DIM = 16

NC = 2
NS = 16
NW = NC * NS
CH = 32
KPW = 320
EPW = KPW * CH
EPAD = NW * EPW
NP = KPW // 2

ND = N + 8
RPW = 624
RTAIL = N - NS * RPW

DEPW = E // NW
DCH = 128
DNCH = DEPW // DCH
DREM = DEPW - DNCH * DCH

F32 = jnp.float32
I32 = jnp.int32


def _mm(a, w):
    return lax.dot_general(a, w, (((1,), (1,)), ((), ())),
                           preferred_element_type=F32)


def _ew_body(ea, w1e, b1, o0, o1, o2):
    outs = (o0, o1, o2)
    for l in range(L):
        outs[l][...] = _mm(ea[...], w1e[l]) + b1[l][None, :]


def _ew_call(edge_attr, W1e, b1):
    BE = 4096
    grid = EPAD // BE
    return pl.pallas_call(
        _ew_body,
        grid=(grid,),
        in_specs=[
            pl.BlockSpec((BE, EDGE_DIM), lambda i: (i, 0)),
            pl.BlockSpec((L, H, EDGE_DIM), lambda i: (0, 0, 0)),
            pl.BlockSpec((L, H), lambda i: (0, 0)),
        ],
        out_specs=[pl.BlockSpec((BE, H), lambda i: (i, 0))] * L,
        out_shape=[jax.ShapeDtypeStruct((EPAD, H), F32)] * L,
    )(edge_attr, W1e, b1)


def _in_body(x, w, b, w1h0, h, hw):
    hv = jnp.maximum(_mm(x[...], w[...]) + b[...], 0.0)
    h[...] = hv
    hw[...] = _mm(hv, w1h0[...])


def _in_call(x, lin_in_W, lin_in_b, W1h0):
    BN = 1000
    return pl.pallas_call(
        _in_body,
        grid=(N // BN,),
        in_specs=[
            pl.BlockSpec((BN, H), lambda i: (i, 0)),
            pl.BlockSpec((H, H), lambda i: (0, 0)),
            pl.BlockSpec((1, H), lambda i: (0, 0)),
            pl.BlockSpec((H, H), lambda i: (0, 0)),
        ],
        out_specs=[pl.BlockSpec((BN, H), lambda i: (i, 0))] * 2,
        out_shape=[jax.ShapeDtypeStruct((N, H), F32)] * 2,
    )(x, lin_in_W, lin_in_b.reshape(1, H), W1h0)


def _upd_body(sp, dp, h, w2, b2, wih, whh, bih, bhh, scale, shift, w1hn,
              hout, hwn):
    s = sp[0] + sp[1]
    deg = dp[0, :, 0:1] + dp[1, :, 0:1]
    agg = _mm(s, w2[...]) + deg * b2[...]
    gi = _mm(agg, wih[...]) + bih[...]
    gh = _mm(h[...], whh[...]) + bhh[...]
    r = jax.nn.sigmoid(gi[:, :H] + gh[:, :H])
    zg = jax.nn.sigmoid(gi[:, H:2 * H] + gh[:, H:2 * H])
    ng = jnp.tanh(gi[:, 2 * H:] + r * gh[:, 2 * H:])
    h_new = (1.0 - zg) * ng + zg * h[...]
    hv = h[...] + h_new * scale[...] + shift[...]
    hout[...] = hv
    hwn[...] = _mm(hv, w1hn[...])


def _upd_call(s_parts, deg_parts, h, W2, b2, Wih, Whh, bih, bhh, scale,
              shift, W1h_next):
    BN = 1000
    return pl.pallas_call(
        _upd_body,
        grid=(N // BN,),
        in_specs=[
            pl.BlockSpec((NC, BN, H), lambda i: (0, i, 0)),
            pl.BlockSpec((NC, BN, EDGE_DIM), lambda i: (0, i, 0)),
            pl.BlockSpec((BN, H), lambda i: (i, 0)),
            pl.BlockSpec((H, H), lambda i: (0, 0)),
            pl.BlockSpec((1, H), lambda i: (0, 0)),
            pl.BlockSpec((3 * H, H), lambda i: (0, 0)),
            pl.BlockSpec((3 * H, H), lambda i: (0, 0)),
            pl.BlockSpec((1, 3 * H), lambda i: (0, 0)),
            pl.BlockSpec((1, 3 * H), lambda i: (0, 0)),
            pl.BlockSpec((1, H), lambda i: (0, 0)),
            pl.BlockSpec((1, H), lambda i: (0, 0)),
            pl.BlockSpec((H, H), lambda i: (0, 0)),
        ],
        out_specs=[pl.BlockSpec((BN, H), lambda i: (i, 0))] * 2,
        out_shape=[jax.ShapeDtypeStruct((N, H), F32)] * 2,
    )(s_parts, deg_parts, h, W2, b2.reshape(1, H), Wih, Whh,
      bih.reshape(1, 3 * H), bhh.reshape(1, 3 * H), scale.reshape(1, H),
      shift.reshape(1, H), W1h_next)


def _msg_body(hw_hbm, ew_hbm, src_hbm, dst_hbm, out_hbm,
              src_all, d0, d1, d2, d3, rows0, rows1, ewb0, ewb1, t0, t1,
              s_shared, gsem0, gsem1, tsem0, tsem1, isem0, isem1, isem2,
              isem3):
    cid = lax.axis_index("c")
    sid = lax.axis_index("s")
    w = sid * NC + cid
    base0 = w * EPW
    krow0 = w * (KPW // 4)

    dring = (d0, d1, d2, d3)
    isems = (isem0, isem1, isem2, isem3)
    sets = ((rows0, ewb0, t0, gsem0, tsem0), (rows1, ewb1, t1, gsem1, tsem1))

    def _z(i, _):
        for j in range(8):
            t0[i, pl.ds(j * 16, 16)] = jnp.zeros((16,), F32)
            t1[i, pl.ds(j * 16, 16)] = jnp.zeros((16,), F32)
        return 0
    lax.fori_loop(0, CH, _z, 0)
    for q in range(19):
        pltpu.sync_copy(t0.at[pl.ds(0, 32)],
                        s_shared.at[pl.ds(sid * RPW + q * 32, 32)])
    pltpu.sync_copy(t0.at[pl.ds(0, 16)],
                    s_shared.at[pl.ds(sid * RPW + 608, 16)])

    @pl.when(sid == NS - 1)
    def _ztail():
        pltpu.sync_copy(t0.at[pl.ds(0, ND - NS * RPW)],
                        s_shared.at[pl.ds(NS * RPW, ND - NS * RPW)])

    pltpu.sync_copy(src_hbm.at[pl.ds(krow0, KPW // 4)], src_all)
    plsc.subcore_barrier()

    def _idxref(k):
        return src_all.at[k // 4, pl.ds((k % 4) * CH, CH)]

    def _dfire(k, slot):
        pltpu.async_copy(dst_hbm.at[pl.ds(base0 + k * CH, CH)],
                         dring[slot], isems[slot])

    def _dwait(k, slot):
        pltpu.make_async_copy(dst_hbm.at[pl.ds(base0 + k * CH, CH)],
                              dring[slot], isems[slot]).wait()

    def _fire(k, st):
        pltpu.async_copy(hw_hbm.at[_idxref(k)], st[0], st[3])
        pltpu.async_copy(ew_hbm.at[pl.ds(base0 + k * CH, CH)], st[1], st[3])

    def _wait(k, st):
        pltpu.make_async_copy(hw_hbm.at[_idxref(k)], st[0], st[3]).wait()
        pltpu.make_async_copy(ew_hbm.at[pl.ds(base0 + k * CH, CH)],
                              st[1], st[3]).wait()

    def _compute(st):
        rbuf, ebuf, tbuf = st[0], st[1], st[2]

        def _row(i, _):
            for j in range(8):
                sl = pl.ds(j * 16, 16)
                tbuf[i, sl] = jnp.maximum(rbuf[i, sl] + ebuf[i, sl], 0.0)
            return 0
        lax.fori_loop(0, CH, _row, 0)

    def _scat_fire(slot, st):
        pltpu.async_copy(st[2], s_shared.at[dring[slot]], st[4], add=True)

    def _scat_wait(slot, st):
        pltpu.make_async_copy(st[2], s_shared.at[dring[slot]], st[4]).wait()

    for k in range(4):
        _dfire(k, k)
    _fire(0, sets[0])
    _fire(1, sets[1])

    def _quad(q, _):
        c0 = 4 * q
        for ci in range(4):
            c = c0 + ci
            st = sets[ci % 2]
            _wait(c, st)
            prev = c - 2
            pslot = (ci - 2) % 4

            @pl.when(prev >= 0)
            def _sw():
                _scat_wait(pslot, st)

                @pl.when(prev + 4 < KPW)
                def _df():
                    _dfire(prev + 4, pslot)
            _compute(st)
            _dwait(c, ci)
            _scat_fire(ci, st)

            nxt = c + 2

            @pl.when(nxt < KPW)
            def _gf():
                _fire(nxt, st)
        return 0
    lax.fori_loop(0, KPW // 4, _quad, 0)

    _scat_wait(2, sets[0])
    _scat_wait(3, sets[1])

    plsc.subcore_barrier()
    pltpu.sync_copy(s_shared.at[pl.ds(sid * RPW, RPW)],
                    out_hbm.at[cid, pl.ds(sid * RPW, RPW)])

    @pl.when(sid == NS - 1)
    def _ftail():
        pltpu.sync_copy(s_shared.at[pl.ds(NS * RPW, RTAIL)],
                        out_hbm.at[cid, pl.ds(NS * RPW, RTAIL)])


def _msg_call(hw, ew_l, src2d, dst1d):
    mesh = plsc.VectorSubcoreMesh(core_axis_name="c", subcore_axis_name="s")
    fn = pl.kernel(
        _msg_body,
        out_type=jax.ShapeDtypeStruct((NC, N, H), F32),
        mesh=mesh,
        scratch_types=[
            pltpu.VMEM((KPW // 4, 128), I32),
            pltpu.VMEM((CH,), I32),
            pltpu.VMEM((CH,), I32),
            pltpu.VMEM((CH,), I32),
            pltpu.VMEM((CH,), I32),
            pltpu.VMEM((CH, H), F32),
            pltpu.VMEM((CH, H), F32),
            pltpu.VMEM((CH, H), F32),
            pltpu.VMEM((CH, H), F32),
            pltpu.VMEM((CH, H), F32),
            pltpu.VMEM((CH, H), F32),
            pltpu.VMEM_SHARED((ND, H), F32),
            pltpu.SemaphoreType.DMA,
            pltpu.SemaphoreType.DMA,
            pltpu.SemaphoreType.DMA,
            pltpu.SemaphoreType.DMA,
            pltpu.SemaphoreType.DMA,
            pltpu.SemaphoreType.DMA,
            pltpu.SemaphoreType.DMA,
            pltpu.SemaphoreType.DMA,
        ],
    )
    return fn(hw, ew_l, src2d, dst1d)


def _deg_body(dst_hbm, out_hbm, dst_v, ones_v, dst16, d_shared):
    cid = lax.axis_index("c")
    sid = lax.axis_index("s")

    def _zz(q, _):
        ones_v[q, pl.ds(0, 16)] = jnp.zeros((16,), F32)
        return 0
    lax.fori_loop(0, DCH, _zz, 0)
    for t in range(6):
        pltpu.sync_copy(ones_v.at[pl.ds(0, 104)],
                        d_shared.at[pl.ds(sid * RPW + t * 104, 104)])

    @pl.when(sid == NS - 1)
    def _ztail():
        pltpu.sync_copy(ones_v.at[pl.ds(0, RTAIL)],
                        d_shared.at[pl.ds(NS * RPW, RTAIL)])
    plsc.subcore_barrier()

    def _o(q, _):
        ones_v[q, pl.ds(0, 16)] = jnp.ones((16,), F32)
        return 0
    lax.fori_loop(0, DCH, _o, 0)

    base0 = (sid * NC + cid) * DEPW

    def _chunk(k, _):
        base = base0 + k * DCH
        pltpu.sync_copy(dst_hbm.at[pl.ds(base, DCH)], dst_v)
        pltpu.sync_copy(ones_v, d_shared.at[dst_v], add=True)
        return 0
    lax.fori_loop(0, DNCH, _chunk, 0)

    base = base0 + DNCH * DCH
    pltpu.sync_copy(dst_hbm.at[pl.ds(base, DREM)], dst16)
    pltpu.sync_copy(ones_v.at[pl.ds(0, DREM)], d_shared.at[dst16], add=True)

    plsc.subcore_barrier()
    pltpu.sync_copy(d_shared.at[pl.ds(sid * RPW, RPW)],
                    out_hbm.at[cid, pl.ds(sid * RPW, RPW)])

    @pl.when(sid == NS - 1)
    def _ftail():
        pltpu.sync_copy(d_shared.at[pl.ds(NS * RPW, RTAIL)],
                        out_hbm.at[cid, pl.ds(NS * RPW, RTAIL)])


def _deg_call(dst):
    mesh = plsc.VectorSubcoreMesh(core_axis_name="c", subcore_axis_name="s")
    fn = pl.kernel(
        _deg_body,
        out_type=jax.ShapeDtypeStruct((NC, N, EDGE_DIM), F32),
        mesh=mesh,
        scratch_types=[
            pltpu.VMEM((DCH,), I32),
            pltpu.VMEM((DCH, EDGE_DIM), F32),
            pltpu.VMEM((DREM,), I32),
            pltpu.VMEM_SHARED((N, EDGE_DIM), F32),
        ],
    )
    return fn(dst)


def _read_body(h, brow, bcol, row, rob, z, pm_a, pm_b):
    pm_a[...] = h[...]
    bc = bcol[...]
    bufs = (pm_a, pm_b)
    for k in range(14):
        s = 1 << k
        av = bufs[k % 2][...]
        cand = jnp.maximum(av[s:], av[:-s])
        sel = jnp.where(bc[s:] == bc[:-s], cand, av[s:])
        bufs[(k + 1) % 2][...] = jnp.concatenate([av[:s], sel], axis=0)

    br = brow[...]
    last = jnp.concatenate(
        [(br[:, 1:] != br[:, :-1]).astype(F32), jnp.ones((1, 1), F32)],
        axis=1)
    gids = lax.broadcasted_iota(I32, (NG, 1), 0)
    hv = h[...]
    pmv = pm_a[...]
    CN = 1000
    zs = jnp.zeros((NG, H), F32)
    zm = jnp.zeros((NG, H), F32)
    cnt = jnp.zeros((NG, 1), F32)
    for c in range(N // CN):
        sl = slice(c * CN, (c + 1) * CN)
        mc = (br[:, sl] == gids).astype(F32)
        zs = zs + jnp.dot(mc, hv[sl], preferred_element_type=F32)
        ml = mc * last[:, sl]
        zm = zm + jnp.dot(ml, pmv[sl], preferred_element_type=F32)
        cnt = cnt + jnp.sum(mc, axis=1, keepdims=True)
    mean = zs / jnp.maximum(cnt, 1.0)
    zmax = jnp.where(cnt > 0, zm, 0.0)
    cat = jnp.concatenate([mean, zmax], axis=1)
    z[...] = jnp.maximum(_mm(cat, row[...]) + rob[...], 0.0)


def _read_call(h, batch, ro_W, ro_b):
    return pl.pallas_call(
        _read_body,
        grid=(1,),
        in_specs=[
            pl.BlockSpec((N, H), lambda i: (0, 0)),
            pl.BlockSpec((1, N), lambda i: (0, 0)),
            pl.BlockSpec((N, 1), lambda i: (0, 0)),
            pl.BlockSpec((H, 2 * H), lambda i: (0, 0)),
            pl.BlockSpec((1, H), lambda i: (0, 0)),
        ],
        out_specs=pl.BlockSpec((NG, H), lambda i: (0, 0)),
        out_shape=jax.ShapeDtypeStruct((NG, H), F32),
        scratch_shapes=[
            pltpu.VMEM((N, H), F32),
            pltpu.VMEM((N, H), F32),
        ],
    )(h, batch.reshape(1, N), batch.reshape(N, 1), ro_W, ro_b.reshape(1, H))


def kernel(x, edge_index, edge_attr, batch, n_graphs, lin_in_W, lin_in_b,
           msg_W1, msg_b1, msg_W2, msg_b2, bn_gamma, bn_beta, bn_mean,
           bn_var, gru_Wih, gru_Whh, gru_bih, gru_bhh, ro_W, ro_b):
    src = edge_index[0]
    dst = edge_index[1]
    W1h = msg_W1[:, :, :H]
    W1e = msg_W1[:, :, H:]

    npad = EPAD - E
    ea_pad = jnp.concatenate(
        [edge_attr, jnp.zeros((npad, EDGE_DIM), F32)], axis=0)
    src2d = jnp.concatenate(
        [src, jnp.zeros((npad,), I32)]).reshape(EPAD // 128, 128)
    dst1d = jnp.concatenate([dst, jnp.full((npad,), N, I32)])

    ew = _ew_call(ea_pad, W1e, msg_b1)
    h, hw = _in_call(x, lin_in_W, lin_in_b, W1h[0])
    deg_parts = _deg_call(dst)

    inv = 1.0 / jnp.sqrt(bn_var + 1e-5)
    scales = bn_gamma * inv
    shifts = bn_beta - bn_mean * scales

    for l in range(L):
        s_parts = _msg_call(hw, ew[l], src2d, dst1d)
        h, hw = _upd_call(s_parts, deg_parts, h, msg_W2[l], msg_b2[l],
                          gru_Wih[l], gru_Whh[l], gru_bih[l], gru_bhh[l],
                          scales[l], shifts[l], W1h[(l + 1) % L])

    return _read_call(h, batch, ro_W, ro_b)
